# baseline (speedup 1.0000x reference)
import functools

import numpy as np
import jax
import jax.numpy as jnp
from jax import lax
from jax.experimental import pallas as pl
from jax.experimental.pallas import tpu as pltpu
from jax.experimental.pallas import tpu_sc as plsc

N = 10000
CB = 2048
RB = 400
MB = 2000
NCB = 5
NRB = 25
NPAD = NCB * CB
NGRP = 625
GW = NCB * 128
NW = 32
RPW = 320
NRE = NW * RPW
GCAP = 8192
ECAP = 8192
RA = 632
NR_ACC = 16 * RA
PAD_DST = 10008
DP = 128

_SP = np.zeros((CB, 128), np.float32)
_SP[np.arange(CB), np.arange(CB) // 16] = 2.0 ** (np.arange(CB) % 16)

_SC_MESH = plsc.VectorSubcoreMesh(core_axis_name="c", subcore_axis_name="s")


def _prep_kernel(a_ref, sp_ref, deg_ref, g_ref):
    cb = pl.program_id(0)
    rb = pl.program_id(1)
    a = a_ref[...]

    @pl.when(rb == 0)
    def _():
        deg_ref[...] = jnp.zeros_like(deg_ref)

    deg_ref[...] += jnp.sum(a, axis=0, keepdims=True)

    dn = (((1,), (0,)), ((), ()))

    @pl.when(cb != NCB - 1)
    def _():
        g_ref[...] = lax.dot_general(a, sp_ref[...], dn,
                                     preferred_element_type=jnp.float32)

    @pl.when(cb == NCB - 1)
    def _():
        lim = N - (NCB - 1) * CB
        ci = lax.broadcasted_iota(jnp.int32, a.shape, 1)
        am = jnp.where(ci < lim, a, 0.0)
        g_ref[...] = lax.dot_general(am, sp_ref[...], dn,
                                     preferred_element_type=jnp.float32)


def _prep(a):
    return pl.pallas_call(
        _prep_kernel,
        grid=(NCB, NRB),
        in_specs=[
            pl.BlockSpec((RB, CB), lambda cb, rb: (rb, cb)),
            pl.BlockSpec((CB, 128), lambda cb, rb: (0, 0)),
        ],
        out_specs=[
            pl.BlockSpec((1, CB), lambda cb, rb: (0, cb)),
            pl.BlockSpec((RB, 128), lambda cb, rb: (rb, cb)),
        ],
        out_shape=[
            jax.ShapeDtypeStruct((1, NPAD), jnp.float32),
            jax.ShapeDtypeStruct((NRE, GW), jnp.float32),
        ],
    )(a, jnp.asarray(_SP))


def _extract_body(g_hbm, esrc_hbm, edst_hbm, ecnt_hbm,
                  gbuf, glist, gbits, esrc_v, edst_v, cntv):
    wid = lax.axis_index("s") * 2 + lax.axis_index("c")
    row0 = wid * RPW
    iota = lax.iota(jnp.int32, 16)
    z16 = jnp.zeros((16,), jnp.int32)
    pad16 = jnp.full((16,), PAD_DST, jnp.int32)

    def fill(i, _):
        esrc_v[pl.ds(i * 16, 16)] = z16
        edst_v[pl.ds(i * 16, 16)] = pad16
        return 0

    lax.fori_loop(0, (ECAP + 128) // 16, fill, 0)

    goff = jnp.zeros((16,), jnp.int32)
    for ck in range(5):
        pltpu.sync_copy(g_hbm.at[pl.ds(row0 + ck * 64, 64)], gbuf)

        def row_body(r, goff, ck=ck):
            rg = row0 + ck * 64 + r
            rgv = lax.broadcast_in_dim(rg, (16,), ())
            gidbase = rg * NGRP
            for vb in range(40):
                gv = gbuf[r, pl.ds(vb * 16, 16)]
                m = gv > 0.0
                if vb == 39:
                    m = m & (iota < 1)
                m = m & (rgv < N)
                cs = lax.cumsum(jnp.where(m, 1, 0), axis=0)
                addr = goff + cs - 1
                am = m & (addr < GCAP)
                gids = lax.broadcast_in_dim(gidbase + vb * 16, (16,), ()) + iota
                plsc.store_scatter(glist, [addr], gids, mask=am)
                plsc.store_scatter(gbits, [addr],
                                   lax.convert_element_type(gv, jnp.int32),
                                   mask=am)
                goff = goff + plsc.all_reduce_population_count(m)
            return goff

        goff = lax.fori_loop(0, 64, row_body, goff)

    goffc = jnp.minimum(goff, GCAP)
    gcnt = jnp.sum(goffc) // 16

    def rec(k, eoff):
        pv = lax.broadcast_in_dim(k, (16,), ())
        gid_b = plsc.load_gather(glist, [pv])
        v_b = plsc.load_gather(gbits, [pv])
        row_b = gid_b // NGRP
        jb_b = (gid_b - row_b * NGRP) * 16
        m = ((lax.shift_right_logical(v_b, iota)) & 1) == 1
        m = m & (pv < goffc)
        cs = lax.cumsum(jnp.where(m, 1, 0), axis=0)
        addr = eoff + cs - 1
        am = m & (addr < ECAP)
        plsc.store_scatter(esrc_v, [addr], row_b, mask=am)
        plsc.store_scatter(edst_v, [addr], jb_b + iota, mask=am)
        return eoff + plsc.all_reduce_population_count(m)

    eoff = lax.fori_loop(0, gcnt, rec, jnp.zeros((16,), jnp.int32))

    cntv[...] = jnp.minimum(eoff, ECAP)
    pltpu.sync_copy(esrc_v.at[pl.ds(0, ECAP)], esrc_hbm.at[pl.ds(wid * ECAP, ECAP)])
    pltpu.sync_copy(edst_v.at[pl.ds(0, ECAP)], edst_hbm.at[pl.ds(wid * ECAP, ECAP)])
    pltpu.sync_copy(cntv, ecnt_hbm.at[pl.ds(wid * 16, 16)])


def _extract(g):
    k = pl.kernel(
        _extract_body,
        out_type=(
            jax.ShapeDtypeStruct((NW * ECAP,), jnp.int32),
            jax.ShapeDtypeStruct((NW * ECAP,), jnp.int32),
            jax.ShapeDtypeStruct((2 * NW * 16,), jnp.int32),
        ),
        mesh=_SC_MESH,
        compiler_params=pltpu.CompilerParams(needs_layout_passes=False),
        scratch_types=[
            pltpu.VMEM((64, GW), jnp.float32),
            pltpu.VMEM((GCAP,), jnp.int32),
            pltpu.VMEM((GCAP,), jnp.int32),
            pltpu.VMEM((ECAP + 128,), jnp.int32),
            pltpu.VMEM((ECAP + 128,), jnp.int32),
            pltpu.VMEM((16,), jnp.int32),
        ],
    )
    return k(g)


def _layer_body(y_hbm, z_hbm, esrc_hbm, edst_hbm, ecnt_hbm, out_hbm,
                sidx, didx, rows, cnt_vm, acc, sem):
    c = lax.axis_index("c")
    s = lax.axis_index("s")
    wid = s * 2 + c

    pltpu.sync_copy(z_hbm.at[pl.ds(s * RA, RA)], acc.at[pl.ds(s * RA, RA)])
    plsc.subcore_barrier()

    pltpu.sync_copy(ecnt_hbm.at[pl.ds(wid * 16, 16)], cnt_vm)
    ecnt = jnp.sum(cnt_vm[...]) // 16

    def ch(ci, _):
        pltpu.sync_copy(esrc_hbm.at[pl.ds(wid * ECAP + ci * 128, 128)], sidx)
        pltpu.sync_copy(edst_hbm.at[pl.ds(wid * ECAP + ci * 128, 128)], didx)
        pltpu.async_copy(y_hbm.at[sidx], rows, sem).wait()
        pltpu.sync_copy(rows, acc.at[didx], add=True)
        return 0

    lax.fori_loop(0, (ecnt + 127) // 128, ch, 0)
    plsc.subcore_barrier()
    pltpu.sync_copy(acc.at[pl.ds(s * RA, RA)],
                    out_hbm.at[c, pl.ds(s * RA, RA)])


def _layer(y, esrc, edst, ecnt):
    k = pl.kernel(
        _layer_body,
        out_type=jax.ShapeDtypeStruct((2, NR_ACC, DP), jnp.float32),
        mesh=_SC_MESH,
        compiler_params=pltpu.CompilerParams(needs_layout_passes=False),
        scratch_types=[
            pltpu.VMEM((128,), jnp.int32),
            pltpu.VMEM((128,), jnp.int32),
            pltpu.VMEM((128, DP), jnp.float32),
            pltpu.VMEM((16,), jnp.int32),
            pltpu.VMEM_SHARED((NR_ACC, DP), jnp.float32),
            pltpu.SemaphoreType.DMA,
        ],
    )
    return k(y, jnp.zeros((NR_ACC, DP), jnp.float32), esrc, edst, ecnt)


def _mm_kernel(h_ref, w_ref, deg_ref, y_ref):
    y = lax.dot_general(h_ref[...], w_ref[...], (((1,), (0,)), ((), ())),
                        preferred_element_type=jnp.float32)
    y_ref[...] = lax.rsqrt(deg_ref[...] + 1.0) * y


def _feature_mm(h, w, deg_col):
    m, d_out = h.shape[0], w.shape[1]
    return pl.pallas_call(
        _mm_kernel,
        grid=(m // MB,),
        in_specs=[
            pl.BlockSpec((MB, h.shape[1]), lambda i: (i, 0)),
            pl.BlockSpec(w.shape, lambda i: (0, 0)),
            pl.BlockSpec((MB, 1), lambda i: (i, 0)),
        ],
        out_specs=pl.BlockSpec((MB, d_out), lambda i: (i, 0)),
        out_shape=jax.ShapeDtypeStruct((m, d_out), jnp.float32),
    )(h, w, deg_col)


def _mid_kernel(a0_ref, a1_ref, y1_ref, deg_ref, b_ref, w_ref, y2_ref):
    s = lax.rsqrt(deg_ref[...] + 1.0)
    h = s * (a0_ref[...] + a1_ref[...] + y1_ref[...]) + b_ref[...]
    h = jnp.where(h > 0, h, jnp.exp(h) - 1.0)
    y = lax.dot_general(h, w_ref[...], (((1,), (0,)), ((), ())),
                        preferred_element_type=jnp.float32)
    y2_ref[...] = s * y


def _mid(a0, a1, y1, deg_col, b1, w2):
    d_in, d_out = w2.shape
    return pl.pallas_call(
        _mid_kernel,
        grid=(N // MB,),
        in_specs=[
            pl.BlockSpec((MB, d_in), lambda i: (i, 0)),
            pl.BlockSpec((MB, d_in), lambda i: (i, 0)),
            pl.BlockSpec((MB, d_in), lambda i: (i, 0)),
            pl.BlockSpec((MB, 1), lambda i: (i, 0)),
            pl.BlockSpec((1, d_in), lambda i: (0, 0)),
            pl.BlockSpec((d_in, d_out), lambda i: (0, 0)),
        ],
        out_specs=pl.BlockSpec((MB, d_out), lambda i: (i, 0)),
        out_shape=jax.ShapeDtypeStruct((N, d_out), jnp.float32),
    )(a0, a1, y1, deg_col, b1.reshape(1, d_in), w2)


def _final_kernel(a0_ref, a1_ref, y2_ref, deg_ref, b_ref, out_ref):
    s = lax.rsqrt(deg_ref[...] + 1.0)
    out_ref[...] = s * (a0_ref[...] + a1_ref[...] + y2_ref[...]) + b_ref[...]


def _final(a0, a1, y2, deg_col, b2):
    d = y2.shape[1]
    return pl.pallas_call(
        _final_kernel,
        grid=(N // MB,),
        in_specs=[
            pl.BlockSpec((MB, d), lambda i: (i, 0)),
            pl.BlockSpec((MB, d), lambda i: (i, 0)),
            pl.BlockSpec((MB, d), lambda i: (i, 0)),
            pl.BlockSpec((MB, 1), lambda i: (i, 0)),
            pl.BlockSpec((1, d), lambda i: (0, 0)),
        ],
        out_specs=pl.BlockSpec((MB, d), lambda i: (i, 0)),
        out_shape=jax.ShapeDtypeStruct((N, d), jnp.float32),
    )(a0, a1, y2, deg_col, b2.reshape(1, d))


def kernel(x, adj_matrix, W1, b1, W2, b2):
    a = adj_matrix[0]
    deg, g = _prep(a)
    deg_col = deg.reshape(NPAD, 1)[:N]
    esrc, edst, ecnt = _extract(g)
    w1p = jnp.pad(W1, ((0, 0), (0, DP - W1.shape[1])))
    w2p = jnp.pad(W2, ((0, 0), (0, DP - W2.shape[1])))
    y1 = _feature_mm(x[0], w1p, deg_col)
    acc1 = _layer(y1, esrc, edst, ecnt)
    y2 = _mid(acc1[0, :N, :64], acc1[1, :N, :64], y1[:, :64],
              deg_col, b1, w2p)
    acc2 = _layer(y2, esrc, edst, ecnt)
    out = _final(acc2[0, :N, :32], acc2[1, :N, :32], y2[:, :32],
                 deg_col, b2)
    return out.reshape(1, N, 32)


# speedup vs baseline: 1.8709x; 1.8709x over previous
import functools

import numpy as np
import jax
import jax.numpy as jnp
from jax import lax
from jax.experimental import pallas as pl
from jax.experimental.pallas import tpu as pltpu
from jax.experimental.pallas import tpu_sc as plsc

N = 10000
CB = 2048
RB = 400
MB = 2000
NCB = 5
NRB = 25
NPAD = NCB * CB
NGRP = 625
GW = NCB * 128
NW = 32
RPW = 320
NRE = NW * RPW
GCAP = 8192
ECAP = 8192
RA = 632
NR_ACC = 16 * RA
PAD_DST = 10008
DP = 128
SLOTG = GCAP // 16
SLOTE = ECAP // 16
SUP = 8

_SP = np.zeros((CB, 128), np.float32)
_SP[np.arange(CB), np.arange(CB) // 16] = 2.0 ** (np.arange(CB) % 16)

_SC_MESH = plsc.VectorSubcoreMesh(core_axis_name="c", subcore_axis_name="s")


def _prep_kernel(a_ref, sp_ref, ones_ref, deg_ref, g_ref):
    cb = pl.program_id(0)
    rb = pl.program_id(1)
    a = a_ref[...]

    @pl.when(rb == 0)
    def _():
        deg_ref[...] = jnp.zeros_like(deg_ref)

    dn = (((1,), (0,)), ((), ()))

    @pl.when(cb != NCB - 1)
    def _():
        ab = a.astype(jnp.bfloat16)
        deg_ref[...] += lax.dot_general(ones_ref[...], ab, dn,
                                        preferred_element_type=jnp.float32)
        g_ref[...] = lax.dot_general(ab, sp_ref[...], dn,
                                     preferred_element_type=jnp.float32)

    @pl.when(cb == NCB - 1)
    def _():
        lim = N - (NCB - 1) * CB
        ci = lax.broadcasted_iota(jnp.int32, a.shape, 1)
        ab = jnp.where(ci < lim, a, 0.0).astype(jnp.bfloat16)
        deg_ref[...] += lax.dot_general(ones_ref[...], ab, dn,
                                        preferred_element_type=jnp.float32)
        g_ref[...] = lax.dot_general(ab, sp_ref[...], dn,
                                     preferred_element_type=jnp.float32)


def _prep(a):
    return pl.pallas_call(
        _prep_kernel,
        grid=(NCB, NRB),
        in_specs=[
            pl.BlockSpec((RB, CB), lambda cb, rb: (rb, cb)),
            pl.BlockSpec((CB, 128), lambda cb, rb: (0, 0)),
            pl.BlockSpec((1, RB), lambda cb, rb: (0, 0)),
        ],
        out_specs=[
            pl.BlockSpec((1, CB), lambda cb, rb: (0, cb)),
            pl.BlockSpec((RB, 128), lambda cb, rb: (rb, cb)),
        ],
        out_shape=[
            jax.ShapeDtypeStruct((1, NPAD), jnp.float32),
            jax.ShapeDtypeStruct((NRE, GW), jnp.float32),
        ],
    )(a, jnp.asarray(_SP, jnp.bfloat16), jnp.ones((1, RB), jnp.bfloat16))


def _lane_bcast(v, l):
    return lax.gather(
        v, jnp.full((16, 1), l, jnp.int32),
        lax.GatherDimensionNumbers(offset_dims=(), collapsed_slice_dims=(0,),
                                   start_index_map=(0,)),
        (1,), mode=lax.GatherScatterMode.PROMISE_IN_BOUNDS)


def _extract_body(g_hbm, esrc_hbm, edst_hbm, ecnt_hbm,
                  gbuf, glist, gbits, esrc_v, edst_v, cntv):
    wid = lax.axis_index("s") * 2 + lax.axis_index("c")
    row0 = wid * RPW
    iota = lax.iota(jnp.int32, 16)
    z16 = jnp.zeros((16,), jnp.int32)
    one16 = jnp.full((16,), 1, jnp.int32)

    def fill(i, _):
        esrc_v[pl.ds(i * 16, 16)] = iota
        edst_v[pl.ds(i * 16, 16)] = (N + (i % 6) * 16) + iota
        return 0

    lax.fori_loop(0, ECAP // 16, fill, 0)

    gcnt = jnp.zeros((16,), jnp.int32)
    for ck in range(5):
        pltpu.sync_copy(g_hbm.at[pl.ds(row0 + ck * 64, 64)], gbuf)

        def row_body(r, gcnt, ck=ck):
            rg = row0 + ck * 64 + r
            rgv = lax.broadcast_in_dim(rg, (16,), ())
            gidbase = rg * NGRP
            for vb in range(40):
                gv = gbuf[r, pl.ds(vb * 16, 16)]
                m = gv > 0.0
                if vb == 39:
                    m = m & (iota < 1)
                m = m & (rgv < N)
                am = m & (gcnt < SLOTG)
                addr = gcnt * 16 + iota
                gids = lax.broadcast_in_dim(gidbase + vb * 16, (16,), ()) + iota
                plsc.store_scatter(glist, [addr], gids, mask=am)
                plsc.store_scatter(gbits, [addr],
                                   lax.convert_element_type(gv, jnp.int32),
                                   mask=am)
                gcnt = gcnt + jnp.where(am, one16, z16)
            return gcnt

        gcnt = lax.fori_loop(0, 64, row_body, gcnt)

    maxg = jnp.max(gcnt)
    gls = [_lane_bcast(gcnt, l) for l in range(16)]

    def blk(q, ecnt):
        gidv = glist[pl.ds(q * 16, 16)]
        bitv = gbits[pl.ds(q * 16, 16)]
        rowv = gidv // NGRP
        jbv = (gidv - rowv * NGRP) * 16
        qv = lax.broadcast_in_dim(q, (16,), ())
        for l in range(16):
            v_b = _lane_bcast(bitv, l)
            row_b = _lane_bcast(rowv, l)
            jb_b = _lane_bcast(jbv, l)
            m = ((lax.shift_right_logical(v_b, iota)) & 1) == 1
            m = m & (qv < gls[l])
            am = m & (ecnt < SLOTE)
            addr = ecnt * 16 + iota
            plsc.store_scatter(esrc_v, [addr], row_b, mask=am)
            plsc.store_scatter(edst_v, [addr], jb_b + iota, mask=am)
            ecnt = ecnt + jnp.where(am, one16, z16)
        return ecnt

    ecnt = lax.fori_loop(0, maxg, blk, jnp.zeros((16,), jnp.int32))

    cntv[...] = jnp.minimum(ecnt, SLOTE)
    pltpu.sync_copy(esrc_v.at[pl.ds(0, ECAP)],
                    esrc_hbm.at[pl.ds(wid * ECAP, ECAP)])
    pltpu.sync_copy(edst_v.at[pl.ds(0, ECAP)],
                    edst_hbm.at[pl.ds(wid * ECAP, ECAP)])
    pltpu.sync_copy(cntv, ecnt_hbm.at[pl.ds(wid * 16, 16)])


def _extract(g):
    k = pl.kernel(
        _extract_body,
        out_type=(
            jax.ShapeDtypeStruct((NW * ECAP,), jnp.int32),
            jax.ShapeDtypeStruct((NW * ECAP,), jnp.int32),
            jax.ShapeDtypeStruct((NW * 16,), jnp.int32),
        ),
        mesh=_SC_MESH,
        compiler_params=pltpu.CompilerParams(needs_layout_passes=False),
        scratch_types=[
            pltpu.VMEM((64, GW), jnp.float32),
            pltpu.VMEM((GCAP,), jnp.int32),
            pltpu.VMEM((GCAP,), jnp.int32),
            pltpu.VMEM((ECAP,), jnp.int32),
            pltpu.VMEM((ECAP,), jnp.int32),
            pltpu.VMEM((16,), jnp.int32),
        ],
    )
    return k(g)


def _layer_body(y_hbm, z_hbm, esrc_hbm, edst_hbm, ecnt_hbm, out_hbm,
                sibuf, dibuf, rows0, rows1, cnt_vm, acc,
                gs0, gs1, ss0, ss1):
    c = lax.axis_index("c")
    s = lax.axis_index("s")
    wid = s * 2 + c

    pltpu.sync_copy(z_hbm.at[pl.ds(s * RA, RA)], acc.at[pl.ds(s * RA, RA)])
    pltpu.sync_copy(esrc_hbm.at[pl.ds(wid * (ECAP // 128), ECAP // 128)],
                    sibuf)
    pltpu.sync_copy(edst_hbm.at[pl.ds(wid * (ECAP // 128), ECAP // 128)],
                    dibuf)
    pltpu.sync_copy(ecnt_hbm.at[pl.ds(wid * 16, 16)], cnt_vm)
    plsc.subcore_barrier()

    maxe = jnp.max(cnt_vm[...])
    npair = (16 * maxe + 255) // 256

    def pair(p, _):
        c0 = p * 2
        c1 = p * 2 + 1
        g0 = pltpu.async_copy(y_hbm.at[sibuf.at[c0]], rows0, gs0)
        g1 = pltpu.async_copy(y_hbm.at[sibuf.at[c1]], rows1, gs1)
        g0.wait()
        s0 = pltpu.async_copy(rows0, acc.at[dibuf.at[c0]], ss0, add=True)
        g1.wait()
        s1 = pltpu.async_copy(rows1, acc.at[dibuf.at[c1]], ss1, add=True)
        s0.wait()
        s1.wait()
        return 0

    lax.fori_loop(0, npair, pair, 0)
    plsc.subcore_barrier()
    pltpu.sync_copy(acc.at[pl.ds(s * RA, RA)],
                    out_hbm.at[c, pl.ds(s * RA, RA)])


def _layer(y, esrc, edst, ecnt):
    d = y.shape[1]
    k = pl.kernel(
        _layer_body,
        out_type=jax.ShapeDtypeStruct((2, NR_ACC, d), jnp.float32),
        mesh=_SC_MESH,
        compiler_params=pltpu.CompilerParams(needs_layout_passes=False,
                                             use_tc_tiling_on_sc=False),
        scratch_types=[
            pltpu.VMEM((ECAP // 128, 128), jnp.int32),
            pltpu.VMEM((ECAP // 128, 128), jnp.int32),
            pltpu.VMEM((128, d), jnp.float32),
            pltpu.VMEM((128, d), jnp.float32),
            pltpu.VMEM((16,), jnp.int32),
            pltpu.VMEM_SHARED((NR_ACC, d), jnp.float32),
            pltpu.SemaphoreType.DMA,
            pltpu.SemaphoreType.DMA,
            pltpu.SemaphoreType.DMA,
            pltpu.SemaphoreType.DMA,
        ],
    )
    return k(y, jnp.zeros((NR_ACC, d), jnp.float32),
             esrc.reshape(NW * ECAP // 128, 128),
             edst.reshape(NW * ECAP // 128, 128), ecnt)


def _mm_kernel(h_ref, w_ref, deg_ref, y_ref):
    y = lax.dot_general(h_ref[...], w_ref[...], (((1,), (0,)), ((), ())),
                        preferred_element_type=jnp.float32)
    y_ref[...] = lax.rsqrt(deg_ref[...] + 1.0) * y


def _feature_mm(h, w, deg_col):
    m, d_out = h.shape[0], w.shape[1]
    return pl.pallas_call(
        _mm_kernel,
        grid=(m // MB,),
        in_specs=[
            pl.BlockSpec((MB, h.shape[1]), lambda i: (i, 0)),
            pl.BlockSpec(w.shape, lambda i: (0, 0)),
            pl.BlockSpec((MB, 1), lambda i: (i, 0)),
        ],
        out_specs=pl.BlockSpec((MB, d_out), lambda i: (i, 0)),
        out_shape=jax.ShapeDtypeStruct((m, d_out), jnp.float32),
    )(h, w, deg_col)


def _mid_kernel(a0_ref, a1_ref, y1_ref, deg_ref, b_ref, w_ref, y2_ref):
    s = lax.rsqrt(deg_ref[...] + 1.0)
    h = s * (a0_ref[...] + a1_ref[...] + y1_ref[...]) + b_ref[...]
    h = jnp.where(h > 0, h, jnp.exp(h) - 1.0)
    y = lax.dot_general(h, w_ref[...], (((1,), (0,)), ((), ())),
                        preferred_element_type=jnp.float32)
    y2_ref[...] = s * y


def _mid(a0, a1, y1, deg_col, b1, w2):
    d_in, d_out = w2.shape
    return pl.pallas_call(
        _mid_kernel,
        grid=(N // MB,),
        in_specs=[
            pl.BlockSpec((MB, d_in), lambda i: (i, 0)),
            pl.BlockSpec((MB, d_in), lambda i: (i, 0)),
            pl.BlockSpec((MB, d_in), lambda i: (i, 0)),
            pl.BlockSpec((MB, 1), lambda i: (i, 0)),
            pl.BlockSpec((1, d_in), lambda i: (0, 0)),
            pl.BlockSpec((d_in, d_out), lambda i: (0, 0)),
        ],
        out_specs=pl.BlockSpec((MB, d_out), lambda i: (i, 0)),
        out_shape=jax.ShapeDtypeStruct((N, d_out), jnp.float32),
    )(a0, a1, y1, deg_col, b1.reshape(1, d_in), w2)


def _final_kernel(a0_ref, a1_ref, y2_ref, deg_ref, b_ref, out_ref):
    s = lax.rsqrt(deg_ref[...] + 1.0)
    out_ref[...] = s * (a0_ref[...] + a1_ref[...] + y2_ref[...]) + b_ref[...]


def _final(a0, a1, y2, deg_col, b2):
    d = y2.shape[1]
    return pl.pallas_call(
        _final_kernel,
        grid=(N // MB,),
        in_specs=[
            pl.BlockSpec((MB, d), lambda i: (i, 0)),
            pl.BlockSpec((MB, d), lambda i: (i, 0)),
            pl.BlockSpec((MB, d), lambda i: (i, 0)),
            pl.BlockSpec((MB, 1), lambda i: (i, 0)),
            pl.BlockSpec((1, d), lambda i: (0, 0)),
        ],
        out_specs=pl.BlockSpec((MB, d), lambda i: (i, 0)),
        out_shape=jax.ShapeDtypeStruct((N, d), jnp.float32),
    )(a0, a1, y2, deg_col, b2.reshape(1, d))


def kernel(x, adj_matrix, W1, b1, W2, b2):
    a = adj_matrix[0]
    deg, g = _prep(a)
    deg_col = deg.reshape(NPAD, 1)[:N]
    esrc, edst, ecnt = _extract(g)
    y1 = _feature_mm(x[0], W1, deg_col)
    acc1 = _layer(y1, esrc, edst, ecnt)
    y2 = _mid(acc1[0, :N], acc1[1, :N], y1, deg_col, b1, W2)
    acc2 = _layer(y2, esrc, edst, ecnt)
    out = _final(acc2[0, :N], acc2[1, :N], y2, deg_col, b2)
    return out.reshape(1, N, 32)
